# baseline (speedup 1.0000x reference)
import functools

import jax
import jax.numpy as jnp
from jax import lax
from jax.experimental import pallas as pl
from jax.experimental.pallas import tpu as pltpu
from jax.experimental.pallas import tpu_sc as plsc

_D = 128


def _sigmoid(v):
    return 1.0 / (1.0 + jnp.exp(-v))


def _silu(v):
    return v * _sigmoid(v)


def _acosh(v):
    return jnp.log(v + jnp.sqrt(v * v - 1.0))


def _cosh_sinhc(n):
    en = jnp.exp(n)
    inv = 1.0 / en
    return 0.5 * (en + inv), 0.5 * (en - inv) / n


def _node_pre_body(x_ref, wlin_ref, bvec_ref, t_ref, x2_ref):
    x = x_ref[...]
    is0 = lax.broadcasted_iota(jnp.int32, x.shape, 1) == 0
    d = _acosh(jnp.clip(x[:, 0:1], 1.0 + 1e-7))
    sp = jnp.where(is0, 0.0, x)
    n = jnp.sqrt(jnp.clip(jnp.sum(sp * sp, axis=1, keepdims=True), 1e-12))
    xt = (d / n) * sp
    y = jnp.dot(xt, wlin_ref[...], preferred_element_type=jnp.float32)
    y = jnp.where(is0, 0.0, y)
    n1 = jnp.sqrt(jnp.clip(jnp.sum(y * y, axis=1, keepdims=True), 1e-12))
    ch1, shc1 = _cosh_sinhc(n1)
    x1 = jnp.where(is0, ch1, shc1 * y)
    bvec = jnp.where(is0, 0.0, bvec_ref[...])
    inner = jnp.sum(x1 * bvec, axis=1, keepdims=True)
    coef = inner / (1.0 + x1[:, 0:1])
    b2 = bvec + coef * (x1 + jnp.where(is0, 1.0, 0.0))
    b20 = b2[:, 0:1]
    nsq2 = jnp.sum(b2 * b2, axis=1, keepdims=True) - 2.0 * b20 * b20
    n2 = jnp.sqrt(jnp.clip(nsq2, 1e-12))
    ch2, shc2 = _cosh_sinhc(n2)
    x2 = ch2 * x1 + shc2 * b2
    d2 = _acosh(jnp.clip(x2[:, 0:1], 1.0 + 1e-7))
    sp2 = jnp.where(is0, 0.0, x2)
    n3 = jnp.sqrt(jnp.clip(jnp.sum(sp2 * sp2, axis=1, keepdims=True), 1e-12))
    xtan = (d2 / n3) * sp2
    hi = lax.bitcast_convert_type(
        x2.astype(jnp.bfloat16).astype(jnp.float32), jnp.int32)
    lo = lax.shift_right_logical(
        lax.bitcast_convert_type(
            xtan.astype(jnp.bfloat16).astype(jnp.float32), jnp.int32), 16)
    t_ref[...] = hi | lo
    x2_ref[...] = x2


def _node_pre(x, W_lin, bias):
    N, D = x.shape
    BN = 1000
    return pl.pallas_call(
        _node_pre_body,
        grid=(N // BN,),
        in_specs=[
            pl.BlockSpec((BN, D), lambda i: (i, 0)),
            pl.BlockSpec((D, D), lambda i: (0, 0)),
            pl.BlockSpec((1, D), lambda i: (0, 0)),
        ],
        out_specs=[pl.BlockSpec((BN, D), lambda i: (i, 0)),
                   pl.BlockSpec((BN, D), lambda i: (i, 0))],
        out_shape=[jax.ShapeDtypeStruct((N, D), jnp.int32),
                   jax.ShapeDtypeStruct((N, D), jnp.float32)],
    )(x, W_lin, bias)


def _edge_body(tr_ref, tc_ref, eattr_ref, emask_ref,
               wa1r_ref, wa1c_ref, wa1e0_ref, wa1e1_ref, ba1_ref,
               wa2_ref, ba2_ref,
               we1x_ref, we1e0_ref, we1e1_ref, be1_ref, we2_ref, be2_ref,
               ea_ref, agg_ref):
    tr = tr_ref[...]
    tc = tc_ref[...]
    himask = jnp.int32(-65536)
    xr = lax.bitcast_convert_type(tr & himask, jnp.float32)
    xtr = lax.bitcast_convert_type(
        lax.shift_left(tr, 16), jnp.float32).astype(jnp.bfloat16)
    xc = lax.bitcast_convert_type(tc & himask, jnp.float32)
    xtc = lax.bitcast_convert_type(
        lax.shift_left(tc, 16), jnp.float32).astype(jnp.bfloat16)
    is0 = lax.broadcasted_iota(jnp.int32, xr.shape, 1) == 0
    p = xr * xc
    s = jnp.sum(p, axis=1, keepdims=True)
    p0 = xr[:, 0:1] * xc[:, 0:1]
    alpha = jnp.clip(2.0 * p0 - s, 1.0 + 1e-7)
    geo = _acosh(alpha)
    eattr = eattr_ref[...]
    ea_ref[...] = jnp.concatenate([eattr, geo], axis=1)
    hA = (jnp.dot(xtr, wa1r_ref[...], preferred_element_type=jnp.float32)
          + jnp.dot(xtc, wa1c_ref[...], preferred_element_type=jnp.float32)
          + eattr * wa1e0_ref[...] + geo * wa1e1_ref[...] + ba1_ref[...])
    sA = _silu(hA).astype(jnp.bfloat16)
    att = _sigmoid(
        jnp.dot(sA, wa2_ref[...], preferred_element_type=jnp.float32)
        + ba2_ref[...]) * emask_ref[...]
    denom = jnp.sqrt(jnp.clip(alpha * alpha - 1.0, 1e-12))
    coef = geo / denom
    c2 = -coef * alpha
    xl0 = coef * xc[:, 0:1] + c2 * xr[:, 0:1]
    coefb = -xl0 / (1.0 + xr[:, 0:1])
    xlf = (coef * xc + (c2 + coefb) * xr
           + jnp.where(is0, coefb, 0.0)).astype(jnp.bfloat16)
    hE = (jnp.dot(xlf, we1x_ref[...], preferred_element_type=jnp.float32)
          + eattr * we1e0_ref[...] + geo * we1e1_ref[...] + be1_ref[...])
    sE = _silu(hE).astype(jnp.bfloat16)
    agg_ref[...] = (jnp.dot(sE, we2_ref[...], preferred_element_type=jnp.float32)
                    + be2_ref[...]) * att


def _edge(Tr, Tc, eattr, emask, Wa1r, Wa1c, wa1e0, wa1e1, ba1, Wa2, ba2,
          We1x, we1e0, we1e1, be1, We2, be2):
    E = Tr.shape[0]
    BE = 1600
    wfull = pl.BlockSpec((_D, _D), lambda i: (0, 0))
    wrow = pl.BlockSpec((1, _D), lambda i: (0, 0))
    return pl.pallas_call(
        _edge_body,
        grid=(E // BE,),
        in_specs=[
            pl.BlockSpec((BE, _D), lambda i: (i, 0)),
            pl.BlockSpec((BE, _D), lambda i: (i, 0)),
            pl.BlockSpec((BE, 1), lambda i: (i, 0)),
            pl.BlockSpec((BE, 1), lambda i: (i, 0)),
            wfull, wfull, wrow, wrow, wrow,
            pl.BlockSpec((_D, 1), lambda i: (0, 0)),
            pl.BlockSpec((1, 1), lambda i: (0, 0)),
            wfull, wrow, wrow, wrow, wfull, wrow,
        ],
        out_specs=[
            pl.BlockSpec((BE, 2), lambda i: (i, 0)),
            pl.BlockSpec((BE, _D), lambda i: (i, 0)),
        ],
        out_shape=[
            jax.ShapeDtypeStruct((E, 2), jnp.float32),
            jax.ShapeDtypeStruct((E, _D), jnp.float32),
        ],
    )(Tr, Tc, eattr, emask, Wa1r, Wa1c, wa1e0, wa1e1, ba1, Wa2, ba2,
      We1x, we1e0, we1e1, be1, We2, be2)


def _node_post_body(x2_ref, *rest):
    p_refs = rest[:-3]
    lng_ref, lnb_ref, out_ref = rest[-3], rest[-2], rest[-1]
    x2 = x2_ref[...]
    is0 = lax.broadcasted_iota(jnp.int32, x2.shape, 1) == 0
    out = p_refs[0][...]
    for pr in p_refs[1:]:
        out = out + pr[...]
    support = jnp.where(is0, 0.0, out)
    inner = jnp.sum(x2 * support, axis=1, keepdims=True)
    coef = inner / (1.0 + x2[:, 0:1])
    supp2 = support + coef * (x2 + jnp.where(is0, 1.0, 0.0))
    s20 = supp2[:, 0:1]
    nsq = jnp.sum(supp2 * supp2, axis=1, keepdims=True) - 2.0 * s20 * s20
    n = jnp.sqrt(jnp.clip(nsq, 1e-12))
    chn, shcn = _cosh_sinhc(n)
    x3 = chn * x2 + shcn * supp2
    d = _acosh(jnp.clip(x3[:, 0:1], 1.0 + 1e-7))
    sp3 = jnp.where(is0, 0.0, x3)
    nsp = jnp.sqrt(jnp.clip(jnp.sum(sp3 * sp3, axis=1, keepdims=True), 1e-12))
    x4 = (d / nsp) * sp3
    mu = jnp.sum(x4, axis=1, keepdims=True) / (_D - 1)
    c = x4 - mu
    var = (jnp.sum(c * c, axis=1, keepdims=True) - mu * mu) / (_D - 1)
    y = c / jnp.sqrt(var + 1e-5) * lng_ref[...] + lnb_ref[...]
    y = jnp.where(is0, 0.0, y)
    y = _silu(y)
    n4 = jnp.sqrt(jnp.clip(jnp.sum(y * y, axis=1, keepdims=True), 1e-12))
    ch4, shc4 = _cosh_sinhc(n4)
    out_ref[...] = jnp.where(is0, ch4, shc4 * y)


def _node_post(x2f, ps, lng, lnb):
    N = x2f.shape[0]
    BN = 1000
    bspec = pl.BlockSpec((BN, _D), lambda i: (i, 0))
    return pl.pallas_call(
        _node_post_body,
        grid=(N // BN,),
        in_specs=[bspec] * (1 + len(ps)) + [
            pl.BlockSpec((1, _D), lambda i: (0, 0)),
            pl.BlockSpec((1, _D), lambda i: (0, 0)),
        ],
        out_specs=bspec,
        out_shape=jax.ShapeDtypeStruct((N, _D), jnp.float32),
    )(x2f, *ps, lng, lnb)


def _sc_gather(T, row, col):
    E = row.shape[0]
    NW = 32
    per = E // NW
    CH = 40
    n_ch = per // CH
    mesh = plsc.VectorSubcoreMesh(core_axis_name="c", subcore_axis_name="s")

    K = 5
    n_outer = n_ch // K
    row3d = row.reshape(NW, n_ch, CH)
    col3d = col.reshape(NW, n_ch, CH)

    @functools.partial(
        pl.kernel,
        mesh=mesh,
        out_type=[
            jax.ShapeDtypeStruct((E, _D), jnp.int32),
            jax.ShapeDtypeStruct((E, _D), jnp.int32),
        ],
        scratch_types=[
            pltpu.VMEM((n_ch, CH), jnp.int32),
            pltpu.VMEM((n_ch, CH), jnp.int32),
            pltpu.VMEM((K * CH, _D), jnp.int32),
            pltpu.VMEM((K * CH, _D), jnp.int32),
            pltpu.SemaphoreType.DMA,
            pltpu.SemaphoreType.DMA,
        ],
    )
    def gk(t_hbm, row_hbm, col_hbm, tr_hbm, tc_hbm,
           idx_r, idx_c, buf_r, buf_c, gsem, ssem):
        wid = lax.axis_index("s") * 2 + lax.axis_index("c")
        base_w = wid * per
        pltpu.sync_copy(row_hbm.at[wid], idx_r)
        pltpu.sync_copy(col_hbm.at[wid], idx_c)

        def body(i, carry):
            off = i * (K * CH)
            gcps = []
            for j in range(K):
                ci = i * K + j
                b = pl.ds(j * CH, CH)
                gcps.append(pltpu.async_copy(
                    t_hbm.at[idx_r.at[ci]], buf_r.at[b], gsem))
                gcps.append(pltpu.async_copy(
                    t_hbm.at[idx_c.at[ci]], buf_c.at[b], gsem))
            for cp in gcps:
                cp.wait()
            scps = []
            for j in range(K):
                o = off + j * CH
                b = pl.ds(j * CH, CH)
                scps.append(pltpu.async_copy(
                    buf_r.at[b], tr_hbm.at[pl.ds(base_w + o, CH)], ssem))
                scps.append(pltpu.async_copy(
                    buf_c.at[b], tc_hbm.at[pl.ds(base_w + o, CH)], ssem))
            for cp in scps:
                cp.wait()
            return carry

        lax.fori_loop(0, n_outer, body, 0)

    return gk(T, row3d, col3d)


def _sc_scatter(agg, row, zeros_nd):
    NP = zeros_nd.shape[0]
    E = row.shape[0]
    per = E // 32
    CH = 40
    n_ch = per // CH
    K = 5
    n_outer = n_ch // K
    rows_per_tile = NP // 16
    mesh = plsc.VectorSubcoreMesh(core_axis_name="c", subcore_axis_name="s")
    row3d = row.reshape(32, n_ch, CH)

    @functools.partial(
        pl.kernel,
        mesh=mesh,
        out_type=jax.ShapeDtypeStruct((2, NP, _D), jnp.float32),
        scratch_types=[
            pltpu.VMEM((K * CH, _D), jnp.float32),
            pltpu.VMEM((n_ch, CH), jnp.int32),
            pltpu.VMEM_SHARED((NP, _D), jnp.float32),
            pltpu.SemaphoreType.DMA,
            pltpu.SemaphoreType.DMA,
        ],
    )
    def sk(agg_hbm, row_hbm, zeros_hbm, out_hbm, buf, idxv, acc, lsem, wsem):
        c = lax.axis_index("c")
        s = lax.axis_index("s")
        wid = c * 16 + s
        pltpu.sync_copy(zeros_hbm.at[pl.ds(s * rows_per_tile, rows_per_tile)],
                        acc.at[pl.ds(s * rows_per_tile, rows_per_tile)])
        pltpu.sync_copy(row_hbm.at[wid], idxv)
        plsc.subcore_barrier()
        base0 = wid * per

        def body(i, carry):
            lcps = []
            for j in range(K):
                lcps.append(pltpu.async_copy(
                    agg_hbm.at[pl.ds(base0 + (i * K + j) * CH, CH)],
                    buf.at[pl.ds(j * CH, CH)], lsem))
            for cp in lcps:
                cp.wait()
            wcps = []
            for j in range(K):
                wcps.append(pltpu.async_copy(
                    buf.at[pl.ds(j * CH, CH)], acc.at[idxv.at[i * K + j]],
                    wsem, add=True))
            for cp in wcps:
                cp.wait()
            return carry

        lax.fori_loop(0, n_outer, body, 0)
        plsc.subcore_barrier()
        pltpu.sync_copy(acc.at[pl.ds(s * rows_per_tile, rows_per_tile)],
                        out_hbm.at[c, pl.ds(s * rows_per_tile, rows_per_tile)])

    return sk(agg, row3d, zeros_nd)


def kernel(x, edge_attr, edges, node_mask, edge_mask, W_lin, bias, W_e1, b_e1,
           W_e2, b_e2, W_a1, b_a1, W_a2, b_a2, ln_g, ln_b):
    N, D = x.shape
    row = edges[0]
    col = edges[1]
    T, x2f = _node_pre(x, W_lin, bias)
    Wa1r = W_a1[:D]
    Wa1c = W_a1[D:2 * D]
    wa1e0 = W_a1[2 * D:2 * D + 1]
    wa1e1 = W_a1[2 * D + 1:]
    ba1 = b_a1.reshape(1, D)
    ba2 = b_a2.reshape(1, 1)
    We1x = W_e1[:D]
    we1e0 = W_e1[D:D + 1]
    we1e1 = W_e1[D + 1:]
    be1 = b_e1.reshape(1, D)
    be2 = b_e2.reshape(1, D)
    bf = jnp.bfloat16
    NP = ((N + 127) // 128) * 128
    zeros_np = jnp.zeros((NP, D), jnp.float32)
    E = row.shape[0]
    NCHUNK = 10
    E2 = E // NCHUNK
    ea_parts, node_parts = [], []
    for ci in range(NCHUNK):
        sl = slice(ci * E2, (ci + 1) * E2)
        Trc, Tcc = _sc_gather(T, row[sl], col[sl])
        ea_c, agg_c = _edge(Trc, Tcc, edge_attr[sl], edge_mask[sl],
                            Wa1r.astype(bf), Wa1c.astype(bf), wa1e0, wa1e1,
                            ba1, W_a2.astype(bf), ba2, We1x.astype(bf),
                            we1e0, we1e1, be1, W_e2.astype(bf), be2)
        p_c = _sc_scatter(agg_c, row[sl], zeros_np)
        ea_parts.append(ea_c)
        node_parts.extend([p_c[0, :N], p_c[1, :N]])
    ea = jnp.concatenate(ea_parts, axis=0)
    lng = jnp.concatenate([jnp.zeros((1,), jnp.float32), ln_g]).reshape(1, D)
    lnb = jnp.concatenate([jnp.zeros((1,), jnp.float32), ln_b]).reshape(1, D)
    xout = _node_post(x2f, node_parts, lng, lnb)
    return (xout, ea)


# speedup vs baseline: 1.0566x; 1.0566x over previous
import functools

import jax
import jax.numpy as jnp
from jax import lax
from jax.experimental import pallas as pl
from jax.experimental.pallas import tpu as pltpu
from jax.experimental.pallas import tpu_sc as plsc

_D = 128


def _sigmoid(v):
    return 1.0 / (1.0 + jnp.exp(-v))


def _silu(v):
    return v * _sigmoid(v)


def _acosh(v):
    return jnp.log(v + jnp.sqrt(v * v - 1.0))


def _cosh_sinhc(n):
    en = jnp.exp(n)
    inv = 1.0 / en
    return 0.5 * (en + inv), 0.5 * (en - inv) / n


def _node_pre_body(x_ref, wlin_ref, bvec_ref, t_ref, x2_ref):
    x = x_ref[...]
    is0 = lax.broadcasted_iota(jnp.int32, x.shape, 1) == 0
    d = _acosh(jnp.clip(x[:, 0:1], 1.0 + 1e-7))
    sp = jnp.where(is0, 0.0, x)
    n = jnp.sqrt(jnp.clip(jnp.sum(sp * sp, axis=1, keepdims=True), 1e-12))
    xt = (d / n) * sp
    y = jnp.dot(xt, wlin_ref[...], preferred_element_type=jnp.float32)
    y = jnp.where(is0, 0.0, y)
    n1 = jnp.sqrt(jnp.clip(jnp.sum(y * y, axis=1, keepdims=True), 1e-12))
    ch1, shc1 = _cosh_sinhc(n1)
    x1 = jnp.where(is0, ch1, shc1 * y)
    bvec = jnp.where(is0, 0.0, bvec_ref[...])
    inner = jnp.sum(x1 * bvec, axis=1, keepdims=True)
    coef = inner / (1.0 + x1[:, 0:1])
    b2 = bvec + coef * (x1 + jnp.where(is0, 1.0, 0.0))
    b20 = b2[:, 0:1]
    nsq2 = jnp.sum(b2 * b2, axis=1, keepdims=True) - 2.0 * b20 * b20
    n2 = jnp.sqrt(jnp.clip(nsq2, 1e-12))
    ch2, shc2 = _cosh_sinhc(n2)
    x2 = ch2 * x1 + shc2 * b2
    d2 = _acosh(jnp.clip(x2[:, 0:1], 1.0 + 1e-7))
    sp2 = jnp.where(is0, 0.0, x2)
    n3 = jnp.sqrt(jnp.clip(jnp.sum(sp2 * sp2, axis=1, keepdims=True), 1e-12))
    xtan = (d2 / n3) * sp2
    hi = lax.bitcast_convert_type(
        x2.astype(jnp.bfloat16).astype(jnp.float32), jnp.int32)
    lo = lax.shift_right_logical(
        lax.bitcast_convert_type(
            xtan.astype(jnp.bfloat16).astype(jnp.float32), jnp.int32), 16)
    t_ref[...] = hi | lo
    x2_ref[...] = x2


def _node_pre(x, W_lin, bias):
    N, D = x.shape
    BN = 1000
    return pl.pallas_call(
        _node_pre_body,
        grid=(N // BN,),
        in_specs=[
            pl.BlockSpec((BN, D), lambda i: (i, 0)),
            pl.BlockSpec((D, D), lambda i: (0, 0)),
            pl.BlockSpec((1, D), lambda i: (0, 0)),
        ],
        out_specs=[pl.BlockSpec((BN, D), lambda i: (i, 0)),
                   pl.BlockSpec((BN, D), lambda i: (i, 0))],
        out_shape=[jax.ShapeDtypeStruct((N, D), jnp.int32),
                   jax.ShapeDtypeStruct((N, D), jnp.float32)],
    )(x, W_lin, bias)


def _edge_body(tr_ref, tc_ref, eattr_ref, emask_ref,
               wa1r_ref, wa1c_ref, wa1e0_ref, wa1e1_ref, ba1_ref,
               wa2_ref, ba2_ref,
               we1x_ref, we1e0_ref, we1e1_ref, be1_ref, we2_ref, be2_ref,
               ea_ref, agg_ref):
    tr = tr_ref[...]
    tc = tc_ref[...]
    himask = jnp.int32(-65536)
    xr = lax.bitcast_convert_type(tr & himask, jnp.float32)
    xtr = lax.bitcast_convert_type(
        lax.shift_left(tr, 16), jnp.float32).astype(jnp.bfloat16)
    xc = lax.bitcast_convert_type(tc & himask, jnp.float32)
    xtc = lax.bitcast_convert_type(
        lax.shift_left(tc, 16), jnp.float32).astype(jnp.bfloat16)
    is0 = lax.broadcasted_iota(jnp.int32, xr.shape, 1) == 0
    p = xr * xc
    s = jnp.sum(p, axis=1, keepdims=True)
    p0 = xr[:, 0:1] * xc[:, 0:1]
    alpha = jnp.clip(2.0 * p0 - s, 1.0 + 1e-7)
    geo = _acosh(alpha)
    eattr = eattr_ref[...]
    ea_ref[...] = jnp.concatenate([eattr, geo], axis=1)
    hA = (jnp.dot(xtr, wa1r_ref[...], preferred_element_type=jnp.float32)
          + jnp.dot(xtc, wa1c_ref[...], preferred_element_type=jnp.float32)
          + eattr * wa1e0_ref[...] + geo * wa1e1_ref[...] + ba1_ref[...])
    sA = _silu(hA).astype(jnp.bfloat16)
    att = _sigmoid(
        jnp.dot(sA, wa2_ref[...], preferred_element_type=jnp.float32)
        + ba2_ref[...]) * emask_ref[...]
    denom = jnp.sqrt(jnp.clip(alpha * alpha - 1.0, 1e-12))
    coef = geo / denom
    c2 = -coef * alpha
    xl0 = coef * xc[:, 0:1] + c2 * xr[:, 0:1]
    coefb = -xl0 / (1.0 + xr[:, 0:1])
    xlf = (coef * xc + (c2 + coefb) * xr
           + jnp.where(is0, coefb, 0.0)).astype(jnp.bfloat16)
    hE = (jnp.dot(xlf, we1x_ref[...], preferred_element_type=jnp.float32)
          + eattr * we1e0_ref[...] + geo * we1e1_ref[...] + be1_ref[...])
    sE = _silu(hE).astype(jnp.bfloat16)
    agg_ref[...] = (jnp.dot(sE, we2_ref[...], preferred_element_type=jnp.float32)
                    + be2_ref[...]) * att


def _edge(Tr, Tc, eattr, emask, Wa1r, Wa1c, wa1e0, wa1e1, ba1, Wa2, ba2,
          We1x, we1e0, we1e1, be1, We2, be2):
    E = Tr.shape[0]
    BE = 1600
    wfull = pl.BlockSpec((_D, _D), lambda i: (0, 0))
    wrow = pl.BlockSpec((1, _D), lambda i: (0, 0))
    return pl.pallas_call(
        _edge_body,
        grid=(E // BE,),
        in_specs=[
            pl.BlockSpec((BE, _D), lambda i: (i, 0)),
            pl.BlockSpec((BE, _D), lambda i: (i, 0)),
            pl.BlockSpec((BE, 1), lambda i: (i, 0)),
            pl.BlockSpec((BE, 1), lambda i: (i, 0)),
            wfull, wfull, wrow, wrow, wrow,
            pl.BlockSpec((_D, 1), lambda i: (0, 0)),
            pl.BlockSpec((1, 1), lambda i: (0, 0)),
            wfull, wrow, wrow, wrow, wfull, wrow,
        ],
        out_specs=[
            pl.BlockSpec((BE, 2), lambda i: (i, 0)),
            pl.BlockSpec((BE, _D), lambda i: (i, 0)),
        ],
        out_shape=[
            jax.ShapeDtypeStruct((E, 2), jnp.float32),
            jax.ShapeDtypeStruct((E, _D), jnp.float32),
        ],
    )(Tr, Tc, eattr, emask, Wa1r, Wa1c, wa1e0, wa1e1, ba1, Wa2, ba2,
      We1x, we1e0, we1e1, be1, We2, be2)


def _node_post_body(x2_ref, *rest):
    p_refs = rest[:-3]
    lng_ref, lnb_ref, out_ref = rest[-3], rest[-2], rest[-1]
    x2 = x2_ref[...]
    is0 = lax.broadcasted_iota(jnp.int32, x2.shape, 1) == 0
    out = p_refs[0][...]
    for pr in p_refs[1:]:
        out = out + pr[...]
    support = jnp.where(is0, 0.0, out)
    inner = jnp.sum(x2 * support, axis=1, keepdims=True)
    coef = inner / (1.0 + x2[:, 0:1])
    supp2 = support + coef * (x2 + jnp.where(is0, 1.0, 0.0))
    s20 = supp2[:, 0:1]
    nsq = jnp.sum(supp2 * supp2, axis=1, keepdims=True) - 2.0 * s20 * s20
    n = jnp.sqrt(jnp.clip(nsq, 1e-12))
    chn, shcn = _cosh_sinhc(n)
    x3 = chn * x2 + shcn * supp2
    d = _acosh(jnp.clip(x3[:, 0:1], 1.0 + 1e-7))
    sp3 = jnp.where(is0, 0.0, x3)
    nsp = jnp.sqrt(jnp.clip(jnp.sum(sp3 * sp3, axis=1, keepdims=True), 1e-12))
    x4 = (d / nsp) * sp3
    mu = jnp.sum(x4, axis=1, keepdims=True) / (_D - 1)
    c = x4 - mu
    var = (jnp.sum(c * c, axis=1, keepdims=True) - mu * mu) / (_D - 1)
    y = c / jnp.sqrt(var + 1e-5) * lng_ref[...] + lnb_ref[...]
    y = jnp.where(is0, 0.0, y)
    y = _silu(y)
    n4 = jnp.sqrt(jnp.clip(jnp.sum(y * y, axis=1, keepdims=True), 1e-12))
    ch4, shc4 = _cosh_sinhc(n4)
    out_ref[...] = jnp.where(is0, ch4, shc4 * y)


def _node_post(x2f, ps, lng, lnb):
    N = x2f.shape[0]
    BN = 1000
    bspec = pl.BlockSpec((BN, _D), lambda i: (i, 0))
    return pl.pallas_call(
        _node_post_body,
        grid=(N // BN,),
        in_specs=[bspec] * (1 + len(ps)) + [
            pl.BlockSpec((1, _D), lambda i: (0, 0)),
            pl.BlockSpec((1, _D), lambda i: (0, 0)),
        ],
        out_specs=bspec,
        out_shape=jax.ShapeDtypeStruct((N, _D), jnp.float32),
    )(x2f, *ps, lng, lnb)


def _sc_gather(T, row, col):
    E = row.shape[0]
    NW = 32
    per = E // NW
    CH = 40
    n_ch = per // CH
    mesh = plsc.VectorSubcoreMesh(core_axis_name="c", subcore_axis_name="s")

    K = 5
    n_outer = n_ch // K
    row3d = row.reshape(NW, n_ch, CH)
    col3d = col.reshape(NW, n_ch, CH)

    @functools.partial(
        pl.kernel,
        mesh=mesh,
        out_type=[
            jax.ShapeDtypeStruct((E, _D), jnp.int32),
            jax.ShapeDtypeStruct((E, _D), jnp.int32),
        ],
        scratch_types=[
            pltpu.VMEM((n_ch, CH), jnp.int32),
            pltpu.VMEM((n_ch, CH), jnp.int32),
            pltpu.VMEM((K * CH, _D), jnp.int32),
            pltpu.VMEM((K * CH, _D), jnp.int32),
            pltpu.SemaphoreType.DMA,
            pltpu.SemaphoreType.DMA,
        ],
    )
    def gk(t_hbm, row_hbm, col_hbm, tr_hbm, tc_hbm,
           idx_r, idx_c, buf_r, buf_c, gsem, ssem):
        wid = lax.axis_index("s") * 2 + lax.axis_index("c")
        base_w = wid * per
        pltpu.sync_copy(row_hbm.at[wid], idx_r)
        pltpu.sync_copy(col_hbm.at[wid], idx_c)

        def body(i, carry):
            off = i * (K * CH)
            gcps = []
            for j in range(K):
                ci = i * K + j
                b = pl.ds(j * CH, CH)
                gcps.append(pltpu.async_copy(
                    t_hbm.at[idx_r.at[ci]], buf_r.at[b], gsem))
                gcps.append(pltpu.async_copy(
                    t_hbm.at[idx_c.at[ci]], buf_c.at[b], gsem))
            for cp in gcps:
                cp.wait()
            scps = []
            for j in range(K):
                o = off + j * CH
                b = pl.ds(j * CH, CH)
                scps.append(pltpu.async_copy(
                    buf_r.at[b], tr_hbm.at[pl.ds(base_w + o, CH)], ssem))
                scps.append(pltpu.async_copy(
                    buf_c.at[b], tc_hbm.at[pl.ds(base_w + o, CH)], ssem))
            for cp in scps:
                cp.wait()
            return carry

        lax.fori_loop(0, n_outer, body, 0)

    return gk(T, row3d, col3d)


def _sc_scatter(agg, row, zeros_nd):
    NP = zeros_nd.shape[0]
    E = row.shape[0]
    per = E // 32
    CH = 40
    n_ch = per // CH
    K = 5
    n_outer = n_ch // K
    rows_per_tile = NP // 16
    mesh = plsc.VectorSubcoreMesh(core_axis_name="c", subcore_axis_name="s")
    row3d = row.reshape(32, n_ch, CH)

    @functools.partial(
        pl.kernel,
        mesh=mesh,
        out_type=jax.ShapeDtypeStruct((2, NP, _D), jnp.float32),
        scratch_types=[
            pltpu.VMEM((K * CH, _D), jnp.float32),
            pltpu.VMEM((n_ch, CH), jnp.int32),
            pltpu.VMEM_SHARED((NP, _D), jnp.float32),
            pltpu.SemaphoreType.DMA,
            pltpu.SemaphoreType.DMA,
        ],
    )
    def sk(agg_hbm, row_hbm, zeros_hbm, out_hbm, buf, idxv, acc, lsem, wsem):
        c = lax.axis_index("c")
        s = lax.axis_index("s")
        wid = c * 16 + s
        pltpu.sync_copy(zeros_hbm.at[pl.ds(s * rows_per_tile, rows_per_tile)],
                        acc.at[pl.ds(s * rows_per_tile, rows_per_tile)])
        pltpu.sync_copy(row_hbm.at[wid], idxv)
        plsc.subcore_barrier()
        base0 = wid * per

        def body(i, carry):
            lcps = []
            for j in range(K):
                lcps.append(pltpu.async_copy(
                    agg_hbm.at[pl.ds(base0 + (i * K + j) * CH, CH)],
                    buf.at[pl.ds(j * CH, CH)], lsem))
            for cp in lcps:
                cp.wait()
            wcps = []
            for j in range(K):
                wcps.append(pltpu.async_copy(
                    buf.at[pl.ds(j * CH, CH)], acc.at[idxv.at[i * K + j]],
                    wsem, add=True))
            for cp in wcps:
                cp.wait()
            return carry

        lax.fori_loop(0, n_outer, body, 0)
        plsc.subcore_barrier()
        pltpu.sync_copy(acc.at[pl.ds(s * rows_per_tile, rows_per_tile)],
                        out_hbm.at[c, pl.ds(s * rows_per_tile, rows_per_tile)])

    return sk(agg, row3d, zeros_nd)


def kernel(x, edge_attr, edges, node_mask, edge_mask, W_lin, bias, W_e1, b_e1,
           W_e2, b_e2, W_a1, b_a1, W_a2, b_a2, ln_g, ln_b):
    N, D = x.shape
    row = edges[0]
    col = edges[1]
    T, x2f = _node_pre(x, W_lin, bias)
    Wa1r = W_a1[:D]
    Wa1c = W_a1[D:2 * D]
    wa1e0 = W_a1[2 * D:2 * D + 1]
    wa1e1 = W_a1[2 * D + 1:]
    ba1 = b_a1.reshape(1, D)
    ba2 = b_a2.reshape(1, 1)
    We1x = W_e1[:D]
    we1e0 = W_e1[D:D + 1]
    we1e1 = W_e1[D + 1:]
    be1 = b_e1.reshape(1, D)
    be2 = b_e2.reshape(1, D)
    bf = jnp.bfloat16
    NP = ((N + 127) // 128) * 128
    zeros_np = jnp.zeros((NP, D), jnp.float32)
    E = row.shape[0]
    NCHUNK = 5
    E2 = E // NCHUNK
    ea_parts, node_parts = [], []
    for ci in range(NCHUNK):
        sl = slice(ci * E2, (ci + 1) * E2)
        Trc, Tcc = _sc_gather(T, row[sl], col[sl])
        ea_c, agg_c = _edge(Trc, Tcc, edge_attr[sl], edge_mask[sl],
                            Wa1r.astype(bf), Wa1c.astype(bf), wa1e0, wa1e1,
                            ba1, W_a2.astype(bf), ba2, We1x.astype(bf),
                            we1e0, we1e1, be1, W_e2.astype(bf), be2)
        p_c = _sc_scatter(agg_c, row[sl], zeros_np)
        ea_parts.append(ea_c)
        node_parts.extend([p_c[0, :N], p_c[1, :N]])
    ea = jnp.concatenate(ea_parts, axis=0)
    lng = jnp.concatenate([jnp.zeros((1,), jnp.float32), ln_g]).reshape(1, D)
    lnb = jnp.concatenate([jnp.zeros((1,), jnp.float32), ln_b]).reshape(1, D)
    xout = _node_post(x2f, node_parts, lng, lnb)
    return (xout, ea)


# speedup vs baseline: 1.1509x; 1.0892x over previous
import functools

import jax
import jax.numpy as jnp
from jax import lax
from jax.experimental import pallas as pl
from jax.experimental.pallas import tpu as pltpu
from jax.experimental.pallas import tpu_sc as plsc

_D = 128


def _sigmoid(v):
    return 1.0 / (1.0 + jnp.exp(-v))


def _silu(v):
    return v * _sigmoid(v)


def _acosh(v):
    return jnp.log(v + jnp.sqrt(v * v - 1.0))


def _cosh_sinhc(n):
    en = jnp.exp(n)
    inv = 1.0 / en
    return 0.5 * (en + inv), 0.5 * (en - inv) / n


def _node_pre_body(x_ref, wlin_ref, bvec_ref, t_ref, x2_ref):
    x = x_ref[...]
    is0 = lax.broadcasted_iota(jnp.int32, x.shape, 1) == 0
    d = _acosh(jnp.clip(x[:, 0:1], 1.0 + 1e-7))
    sp = jnp.where(is0, 0.0, x)
    n = jnp.sqrt(jnp.clip(jnp.sum(sp * sp, axis=1, keepdims=True), 1e-12))
    xt = (d / n) * sp
    y = jnp.dot(xt, wlin_ref[...], preferred_element_type=jnp.float32)
    y = jnp.where(is0, 0.0, y)
    n1 = jnp.sqrt(jnp.clip(jnp.sum(y * y, axis=1, keepdims=True), 1e-12))
    ch1, shc1 = _cosh_sinhc(n1)
    x1 = jnp.where(is0, ch1, shc1 * y)
    bvec = jnp.where(is0, 0.0, bvec_ref[...])
    inner = jnp.sum(x1 * bvec, axis=1, keepdims=True)
    coef = inner / (1.0 + x1[:, 0:1])
    b2 = bvec + coef * (x1 + jnp.where(is0, 1.0, 0.0))
    b20 = b2[:, 0:1]
    nsq2 = jnp.sum(b2 * b2, axis=1, keepdims=True) - 2.0 * b20 * b20
    n2 = jnp.sqrt(jnp.clip(nsq2, 1e-12))
    ch2, shc2 = _cosh_sinhc(n2)
    x2 = ch2 * x1 + shc2 * b2
    d2 = _acosh(jnp.clip(x2[:, 0:1], 1.0 + 1e-7))
    sp2 = jnp.where(is0, 0.0, x2)
    n3 = jnp.sqrt(jnp.clip(jnp.sum(sp2 * sp2, axis=1, keepdims=True), 1e-12))
    xtan = (d2 / n3) * sp2
    hi = lax.bitcast_convert_type(
        x2.astype(jnp.bfloat16).astype(jnp.float32), jnp.int32)
    lo = lax.shift_right_logical(
        lax.bitcast_convert_type(
            xtan.astype(jnp.bfloat16).astype(jnp.float32), jnp.int32), 16)
    t_ref[...] = hi | lo
    x2_ref[...] = x2


def _node_pre(x, W_lin, bias):
    N, D = x.shape
    BN = 1000
    return pl.pallas_call(
        _node_pre_body,
        grid=(N // BN,),
        in_specs=[
            pl.BlockSpec((BN, D), lambda i: (i, 0)),
            pl.BlockSpec((D, D), lambda i: (0, 0)),
            pl.BlockSpec((1, D), lambda i: (0, 0)),
        ],
        out_specs=[pl.BlockSpec((BN, D), lambda i: (i, 0)),
                   pl.BlockSpec((BN, D), lambda i: (i, 0))],
        out_shape=[jax.ShapeDtypeStruct((N, D), jnp.int32),
                   jax.ShapeDtypeStruct((N, D), jnp.float32)],
    )(x, W_lin, bias)


def _edge_body(tr_ref, tc_ref, eattr_ref, emask_ref,
               wa1r_ref, wa1c_ref, wa1e0_ref, wa1e1_ref, ba1_ref,
               wa2_ref, ba2_ref,
               we1x_ref, we1e0_ref, we1e1_ref, be1_ref, we2_ref, be2_ref,
               ea_ref, agg_ref):
    tr = tr_ref[...]
    tc = tc_ref[...]
    himask = jnp.int32(-65536)
    xr = lax.bitcast_convert_type(tr & himask, jnp.float32)
    xtr = lax.bitcast_convert_type(
        lax.shift_left(tr, 16), jnp.float32).astype(jnp.bfloat16)
    xc = lax.bitcast_convert_type(tc & himask, jnp.float32)
    xtc = lax.bitcast_convert_type(
        lax.shift_left(tc, 16), jnp.float32).astype(jnp.bfloat16)
    is0 = lax.broadcasted_iota(jnp.int32, xr.shape, 1) == 0
    p = xr * xc
    s = jnp.sum(p, axis=1, keepdims=True)
    p0 = xr[:, 0:1] * xc[:, 0:1]
    alpha = jnp.clip(2.0 * p0 - s, 1.0 + 1e-7)
    geo = _acosh(alpha)
    eattr = eattr_ref[...]
    ea_ref[...] = jnp.concatenate([eattr, geo], axis=1)
    hA = (jnp.dot(xtr, wa1r_ref[...], preferred_element_type=jnp.float32)
          + jnp.dot(xtc, wa1c_ref[...], preferred_element_type=jnp.float32)
          + eattr * wa1e0_ref[...] + geo * wa1e1_ref[...] + ba1_ref[...])
    sA = _silu(hA).astype(jnp.bfloat16)
    att = _sigmoid(
        jnp.dot(sA, wa2_ref[...], preferred_element_type=jnp.float32)
        + ba2_ref[...]) * emask_ref[...]
    denom = jnp.sqrt(jnp.clip(alpha * alpha - 1.0, 1e-12))
    coef = geo / denom
    c2 = -coef * alpha
    xl0 = coef * xc[:, 0:1] + c2 * xr[:, 0:1]
    coefb = -xl0 / (1.0 + xr[:, 0:1])
    xlf = (coef * xc + (c2 + coefb) * xr
           + jnp.where(is0, coefb, 0.0)).astype(jnp.bfloat16)
    hE = (jnp.dot(xlf, we1x_ref[...], preferred_element_type=jnp.float32)
          + eattr * we1e0_ref[...] + geo * we1e1_ref[...] + be1_ref[...])
    sE = _silu(hE).astype(jnp.bfloat16)
    agg_ref[...] = (jnp.dot(sE, we2_ref[...], preferred_element_type=jnp.float32)
                    + be2_ref[...]) * att


def _edge(Tr, Tc, eattr, emask, Wa1r, Wa1c, wa1e0, wa1e1, ba1, Wa2, ba2,
          We1x, we1e0, we1e1, be1, We2, be2):
    E = Tr.shape[0]
    BE = 1600
    wfull = pl.BlockSpec((_D, _D), lambda i: (0, 0))
    wrow = pl.BlockSpec((1, _D), lambda i: (0, 0))
    return pl.pallas_call(
        _edge_body,
        grid=(E // BE,),
        in_specs=[
            pl.BlockSpec((BE, _D), lambda i: (i, 0)),
            pl.BlockSpec((BE, _D), lambda i: (i, 0)),
            pl.BlockSpec((BE, 1), lambda i: (i, 0)),
            pl.BlockSpec((BE, 1), lambda i: (i, 0)),
            wfull, wfull, wrow, wrow, wrow,
            pl.BlockSpec((_D, 1), lambda i: (0, 0)),
            pl.BlockSpec((1, 1), lambda i: (0, 0)),
            wfull, wrow, wrow, wrow, wfull, wrow,
        ],
        out_specs=[
            pl.BlockSpec((BE, 2), lambda i: (i, 0)),
            pl.BlockSpec((BE, _D), lambda i: (i, 0)),
        ],
        out_shape=[
            jax.ShapeDtypeStruct((E, 2), jnp.float32),
            jax.ShapeDtypeStruct((E, _D), jnp.float32),
        ],
    )(Tr, Tc, eattr, emask, Wa1r, Wa1c, wa1e0, wa1e1, ba1, Wa2, ba2,
      We1x, we1e0, we1e1, be1, We2, be2)


def _node_post_body(x2_ref, *rest):
    p_refs = rest[:-3]
    lng_ref, lnb_ref, out_ref = rest[-3], rest[-2], rest[-1]
    x2 = x2_ref[...]
    is0 = lax.broadcasted_iota(jnp.int32, x2.shape, 1) == 0
    out = p_refs[0][...]
    for pr in p_refs[1:]:
        out = out + pr[...]
    support = jnp.where(is0, 0.0, out)
    inner = jnp.sum(x2 * support, axis=1, keepdims=True)
    coef = inner / (1.0 + x2[:, 0:1])
    supp2 = support + coef * (x2 + jnp.where(is0, 1.0, 0.0))
    s20 = supp2[:, 0:1]
    nsq = jnp.sum(supp2 * supp2, axis=1, keepdims=True) - 2.0 * s20 * s20
    n = jnp.sqrt(jnp.clip(nsq, 1e-12))
    chn, shcn = _cosh_sinhc(n)
    x3 = chn * x2 + shcn * supp2
    d = _acosh(jnp.clip(x3[:, 0:1], 1.0 + 1e-7))
    sp3 = jnp.where(is0, 0.0, x3)
    nsp = jnp.sqrt(jnp.clip(jnp.sum(sp3 * sp3, axis=1, keepdims=True), 1e-12))
    x4 = (d / nsp) * sp3
    mu = jnp.sum(x4, axis=1, keepdims=True) / (_D - 1)
    c = x4 - mu
    var = (jnp.sum(c * c, axis=1, keepdims=True) - mu * mu) / (_D - 1)
    y = c / jnp.sqrt(var + 1e-5) * lng_ref[...] + lnb_ref[...]
    y = jnp.where(is0, 0.0, y)
    y = _silu(y)
    n4 = jnp.sqrt(jnp.clip(jnp.sum(y * y, axis=1, keepdims=True), 1e-12))
    ch4, shc4 = _cosh_sinhc(n4)
    out_ref[...] = jnp.where(is0, ch4, shc4 * y)


def _node_post(x2f, ps, lng, lnb):
    N = x2f.shape[0]
    BN = 1000
    bspec = pl.BlockSpec((BN, _D), lambda i: (i, 0))
    return pl.pallas_call(
        _node_post_body,
        grid=(N // BN,),
        in_specs=[bspec] * (1 + len(ps)) + [
            pl.BlockSpec((1, _D), lambda i: (0, 0)),
            pl.BlockSpec((1, _D), lambda i: (0, 0)),
        ],
        out_specs=bspec,
        out_shape=jax.ShapeDtypeStruct((N, _D), jnp.float32),
    )(x2f, *ps, lng, lnb)


def _sc_gather(T, row, col):
    E = row.shape[0]
    NW = 32
    per = E // NW
    CH = 40
    n_ch = per // CH
    mesh = plsc.VectorSubcoreMesh(core_axis_name="c", subcore_axis_name="s")

    K = 2
    n_outer = n_ch // K
    row3d = row.reshape(NW, n_ch, CH)
    col3d = col.reshape(NW, n_ch, CH)

    @functools.partial(
        pl.kernel,
        mesh=mesh,
        out_type=[
            jax.ShapeDtypeStruct((E, _D), jnp.int32),
            jax.ShapeDtypeStruct((E, _D), jnp.int32),
        ],
        scratch_types=[
            pltpu.VMEM((n_ch, CH), jnp.int32),
            pltpu.VMEM((n_ch, CH), jnp.int32),
            pltpu.VMEM((K * CH, _D), jnp.int32),
            pltpu.VMEM((K * CH, _D), jnp.int32),
            pltpu.VMEM_SHARED((T.shape[0], _D), jnp.int32),
            pltpu.SemaphoreType.DMA,
            pltpu.SemaphoreType.DMA,
        ],
    )
    def gk(t_hbm, row_hbm, col_hbm, tr_hbm, tc_hbm,
           idx_r, idx_c, buf_r, buf_c, tshr, gsem, ssem):
        s = lax.axis_index("s")
        wid = s * 2 + lax.axis_index("c")
        base_w = wid * per
        rows_per = T.shape[0] // 16
        pltpu.sync_copy(t_hbm.at[pl.ds(s * rows_per, rows_per)],
                        tshr.at[pl.ds(s * rows_per, rows_per)])
        pltpu.sync_copy(row_hbm.at[wid], idx_r)
        pltpu.sync_copy(col_hbm.at[wid], idx_c)
        plsc.subcore_barrier()

        def body(i, carry):
            off = i * (K * CH)
            gcps = []
            for j in range(K):
                ci = i * K + j
                b = pl.ds(j * CH, CH)
                gcps.append(pltpu.async_copy(
                    tshr.at[idx_r.at[ci]], buf_r.at[b], gsem))
                gcps.append(pltpu.async_copy(
                    tshr.at[idx_c.at[ci]], buf_c.at[b], gsem))
            for cp in gcps:
                cp.wait()
            scps = []
            for j in range(K):
                o = off + j * CH
                b = pl.ds(j * CH, CH)
                scps.append(pltpu.async_copy(
                    buf_r.at[b], tr_hbm.at[pl.ds(base_w + o, CH)], ssem))
                scps.append(pltpu.async_copy(
                    buf_c.at[b], tc_hbm.at[pl.ds(base_w + o, CH)], ssem))
            for cp in scps:
                cp.wait()
            return carry

        lax.fori_loop(0, n_outer, body, 0)

    return gk(T, row3d, col3d)


def _sc_scatter(agg, row, zeros_nd):
    NP = zeros_nd.shape[0]
    E = row.shape[0]
    per = E // 32
    CH = 40
    n_ch = per // CH
    K = 5
    n_outer = n_ch // K
    rows_per_tile = NP // 16
    mesh = plsc.VectorSubcoreMesh(core_axis_name="c", subcore_axis_name="s")
    row3d = row.reshape(32, n_ch, CH)

    @functools.partial(
        pl.kernel,
        mesh=mesh,
        out_type=jax.ShapeDtypeStruct((2, NP, _D), jnp.float32),
        scratch_types=[
            pltpu.VMEM((K * CH, _D), jnp.float32),
            pltpu.VMEM((n_ch, CH), jnp.int32),
            pltpu.VMEM_SHARED((NP, _D), jnp.float32),
            pltpu.SemaphoreType.DMA,
            pltpu.SemaphoreType.DMA,
        ],
    )
    def sk(agg_hbm, row_hbm, zeros_hbm, out_hbm, buf, idxv, acc, lsem, wsem):
        c = lax.axis_index("c")
        s = lax.axis_index("s")
        wid = c * 16 + s
        pltpu.sync_copy(zeros_hbm.at[pl.ds(s * rows_per_tile, rows_per_tile)],
                        acc.at[pl.ds(s * rows_per_tile, rows_per_tile)])
        pltpu.sync_copy(row_hbm.at[wid], idxv)
        plsc.subcore_barrier()
        base0 = wid * per

        def body(i, carry):
            lcps = []
            for j in range(K):
                lcps.append(pltpu.async_copy(
                    agg_hbm.at[pl.ds(base0 + (i * K + j) * CH, CH)],
                    buf.at[pl.ds(j * CH, CH)], lsem))
            for cp in lcps:
                cp.wait()
            wcps = []
            for j in range(K):
                wcps.append(pltpu.async_copy(
                    buf.at[pl.ds(j * CH, CH)], acc.at[idxv.at[i * K + j]],
                    wsem, add=True))
            for cp in wcps:
                cp.wait()
            return carry

        lax.fori_loop(0, n_outer, body, 0)
        plsc.subcore_barrier()
        pltpu.sync_copy(acc.at[pl.ds(s * rows_per_tile, rows_per_tile)],
                        out_hbm.at[c, pl.ds(s * rows_per_tile, rows_per_tile)])

    return sk(agg, row3d, zeros_nd)


def kernel(x, edge_attr, edges, node_mask, edge_mask, W_lin, bias, W_e1, b_e1,
           W_e2, b_e2, W_a1, b_a1, W_a2, b_a2, ln_g, ln_b):
    N, D = x.shape
    row = edges[0]
    col = edges[1]
    T, x2f = _node_pre(x, W_lin, bias)
    Wa1r = W_a1[:D]
    Wa1c = W_a1[D:2 * D]
    wa1e0 = W_a1[2 * D:2 * D + 1]
    wa1e1 = W_a1[2 * D + 1:]
    ba1 = b_a1.reshape(1, D)
    ba2 = b_a2.reshape(1, 1)
    We1x = W_e1[:D]
    we1e0 = W_e1[D:D + 1]
    we1e1 = W_e1[D + 1:]
    be1 = b_e1.reshape(1, D)
    be2 = b_e2.reshape(1, D)
    bf = jnp.bfloat16
    NP = ((N + 127) // 128) * 128
    zeros_np = jnp.zeros((NP, D), jnp.float32)
    Tp = jnp.pad(T, ((0, NP - N), (0, 0)))
    E = row.shape[0]
    NCHUNK = 5
    E2 = E // NCHUNK
    ea_parts, node_parts = [], []
    for ci in range(NCHUNK):
        sl = slice(ci * E2, (ci + 1) * E2)
        Trc, Tcc = _sc_gather(Tp, row[sl], col[sl])
        ea_c, agg_c = _edge(Trc, Tcc, edge_attr[sl], edge_mask[sl],
                            Wa1r.astype(bf), Wa1c.astype(bf), wa1e0, wa1e1,
                            ba1, W_a2.astype(bf), ba2, We1x.astype(bf),
                            we1e0, we1e1, be1, W_e2.astype(bf), be2)
        p_c = _sc_scatter(agg_c, row[sl], zeros_np)
        ea_parts.append(ea_c)
        node_parts.extend([p_c[0, :N], p_c[1, :N]])
    ea = jnp.concatenate(ea_parts, axis=0)
    lng = jnp.concatenate([jnp.zeros((1,), jnp.float32), ln_g]).reshape(1, D)
    lnb = jnp.concatenate([jnp.zeros((1,), jnp.float32), ln_b]).reshape(1, D)
    xout = _node_post(x2f, node_parts, lng, lnb)
    return (xout, ea)
